# baseline (speedup 1.0000x reference)
import functools
import math

import jax
import jax.numpy as jnp
from jax.experimental import pallas as pl

HEADS = 8
BUCKET = 64
NHASH = 4
ROWBLK = 512


def _ln(x, g, b):
    m = jnp.mean(x, axis=-1, keepdims=True)
    v = jnp.mean(jnp.square(x - m), axis=-1, keepdims=True)
    return (x - m) * jax.lax.rsqrt(v + 1e-5) * g + b


def _qkv_body(x_ref, g_ref, b_ref, wqk_ref, wv_ref, qk_ref, v_ref):
    h = _ln(x_ref[...], g_ref[...], b_ref[...])
    qk_ref[...] = jnp.dot(h, wqk_ref[...], preferred_element_type=jnp.float32)
    v_ref[...] = jnp.dot(h, wv_ref[...], preferred_element_type=jnp.float32)


def _keys_body(qkh_ref, rot_ref, keys_ref, *, S, n_buckets):
    q = qkh_ref[0]
    r = jnp.dot(q, rot_ref[...], preferred_element_type=jnp.float32)
    r = r.reshape(S, NHASH, n_buckets // 2)
    full = jnp.concatenate([r, -r], axis=-1)
    am = jnp.argmax(full, axis=-1).astype(jnp.int32)
    bucket = am + (jnp.arange(NHASH, dtype=jnp.int32) * n_buckets)[None, :]
    keys = (S * bucket).T + jnp.arange(S, dtype=jnp.int32)[None, :]
    keys_ref[...] = keys.reshape(1, 1, NHASH * S)


def _attn_body(sqk_ref, sv_ref, st_ref, so_ref, slse_ref, *, n_chunks, d):
    scale = d ** -0.5

    qk = sqk_ref[0]
    vv = sv_ref[0]
    tt = st_ref[0, 0]
    for c in range(n_chunks):
        p = (c - 1) % n_chunks
        qc = qk[c * BUCKET:(c + 1) * BUCKET]
        kp = qk[p * BUCKET:(p + 1) * BUCKET]
        vc = vv[c * BUCKET:(c + 1) * BUCKET]
        vp = vv[p * BUCKET:(p + 1) * BUCKET]
        tq = tt[c * BUCKET:(c + 1) * BUCKET]
        tp = tt[p * BUCKET:(p + 1) * BUCKET]
        kv = jnp.concatenate([qc, kp], axis=0)
        nrm = jnp.maximum(
            jnp.sqrt(jnp.sum(kv * kv, axis=-1, keepdims=True)), 1e-12)
        k = kv / nrm
        v = jnp.concatenate([vc, vp], axis=0)
        tkv = jnp.concatenate([tq, tp], axis=0)
        dots = jnp.dot(qc, k.T, preferred_element_type=jnp.float32) * scale
        dots = jnp.where(tq[:, None] == tkv[None, :], -5e4, dots)
        m = jnp.max(dots, axis=-1, keepdims=True)
        e = jnp.exp(dots - m)
        s = jnp.sum(e, axis=-1, keepdims=True)
        o = jnp.dot(e / s, v, preferred_element_type=jnp.float32)
        so_ref[0, c * BUCKET:(c + 1) * BUCKET, :] = o
        slse_ref[0, 0, c * BUCKET:(c + 1) * BUCKET] = (m + jnp.log(s))[:, 0]


def _back_body(ao_ref, x1_ref, x2_ref, wo_ref, bo_ref, g_ref, b_ref,
               w1_ref, b1_ref, w2_ref, b2_ref, x1o_ref, x2o_ref):
    x1n = x1_ref[...] + jnp.dot(ao_ref[...], wo_ref[...],
                                preferred_element_type=jnp.float32) + bo_ref[...]
    h = _ln(x1n, g_ref[...], b_ref[...])
    t = jnp.dot(h, w1_ref[...], preferred_element_type=jnp.float32) + b1_ref[...]
    t = 0.5 * t * (1.0 + jax.lax.erf(t * (2.0 ** -0.5)))
    x2o_ref[...] = x2_ref[...] + jnp.dot(
        t, w2_ref[...], preferred_element_type=jnp.float32) + b2_ref[...]
    x1o_ref[...] = x1n


def _qkv_call(x, g, b, wqk, wv):
    R, D = x.shape
    grid = (R // ROWBLK,)
    blk = pl.BlockSpec((ROWBLK, D), lambda i: (i, 0))
    wblk = pl.BlockSpec((D, D), lambda i: (0, 0))
    vblk = pl.BlockSpec((1, D), lambda i: (0, 0))
    return pl.pallas_call(
        _qkv_body,
        grid=grid,
        in_specs=[blk, vblk, vblk, wblk, wblk],
        out_specs=[blk, blk],
        out_shape=[jax.ShapeDtypeStruct((R, D), jnp.float32)] * 2,
    )(x, g.reshape(1, D), b.reshape(1, D), wqk, wv)


def _keys_call(qkh, rot, S, n_buckets):
    bh, _, d = qkh.shape
    return pl.pallas_call(
        functools.partial(_keys_body, S=S, n_buckets=n_buckets),
        grid=(bh,),
        in_specs=[pl.BlockSpec((1, S, d), lambda i: (i, 0, 0)),
                  pl.BlockSpec((d, NHASH * (n_buckets // 2)), lambda i: (0, 0))],
        out_specs=pl.BlockSpec((1, 1, NHASH * S), lambda i: (i, 0, 0)),
        out_shape=jax.ShapeDtypeStruct((bh, 1, NHASH * S), jnp.int32),
    )(qkh, rot)


def _attn_call(sqk, sv, st):
    bh, T, d = sqk.shape
    n_chunks = T // BUCKET
    rblk = pl.BlockSpec((1, T, d), lambda i: (i, 0, 0))
    iblk = pl.BlockSpec((1, 1, T), lambda i: (i, 0, 0))
    return pl.pallas_call(
        functools.partial(_attn_body, n_chunks=n_chunks, d=d),
        grid=(bh,),
        in_specs=[rblk, rblk, iblk],
        out_specs=[rblk, iblk],
        out_shape=[jax.ShapeDtypeStruct((bh, T, d), jnp.float32),
                   jax.ShapeDtypeStruct((bh, 1, T), jnp.float32)],
    )(sqk, sv, st.reshape(bh, 1, T))


def _back_call(ao, x1, x2, p):
    R, D = ao.shape
    FFD = p['W1'].shape[1]
    grid = (R // ROWBLK,)
    blk = pl.BlockSpec((ROWBLK, D), lambda i: (i, 0))
    return pl.pallas_call(
        _back_body,
        grid=grid,
        in_specs=[blk, blk, blk,
                  pl.BlockSpec((D, D), lambda i: (0, 0)),
                  pl.BlockSpec((1, D), lambda i: (0, 0)),
                  pl.BlockSpec((1, D), lambda i: (0, 0)),
                  pl.BlockSpec((1, D), lambda i: (0, 0)),
                  pl.BlockSpec((D, FFD), lambda i: (0, 0)),
                  pl.BlockSpec((1, FFD), lambda i: (0, 0)),
                  pl.BlockSpec((FFD, D), lambda i: (0, 0)),
                  pl.BlockSpec((1, D), lambda i: (0, 0))],
        out_specs=[blk, blk],
        out_shape=[jax.ShapeDtypeStruct((R, D), jnp.float32)] * 2,
    )(ao, x1, x2, p['Wo'], p['bo'].reshape(1, D), p['ln2_g'].reshape(1, D),
      p['ln2_b'].reshape(1, D), p['W1'], p['b1'].reshape(1, FFD),
      p['W2'], p['b2'].reshape(1, D))


def kernel(input_ids, params):
    B, S = input_ids.shape
    D = params['token_emb'].shape[1]
    DH = D // HEADS
    n_buckets = S // BUCKET
    T = NHASH * S

    x = params['token_emb'][input_ids] + params['pos_emb'][:S][None, :, :]
    x1 = x.reshape(B * S, D)
    x2 = x1

    for i, p in enumerate(params['layers']):
        rot_key = jax.random.fold_in(jax.random.key(42), i)
        rot = jax.random.normal(rot_key, (1, DH, NHASH, n_buckets // 2),
                                dtype=jnp.float32)
        rot = rot.reshape(DH, NHASH * (n_buckets // 2))

        qk, v = _qkv_call(x2, p['ln1_g'], p['ln1_b'], p['Wqk'], p['Wv'])
        def heads(t):
            return (t.reshape(B, S, HEADS, DH).transpose(0, 2, 1, 3)
                    .reshape(B * HEADS, S, DH))
        qkh, vh = heads(qk), heads(v)

        keys = _keys_call(qkh, rot, S, n_buckets).reshape(B * HEADS, T)
        sticker = jnp.argsort(keys, axis=-1).astype(jnp.int32)
        undo = jnp.argsort(sticker, axis=-1).astype(jnp.int32)
        st = sticker % S
        sqk = jnp.take_along_axis(qkh, st[..., None], axis=1)
        sv = jnp.take_along_axis(vh, st[..., None], axis=1)

        so, slse = _attn_call(sqk, sv, st)
        o = jnp.take_along_axis(so, undo[..., None], axis=1)
        logits = jnp.take_along_axis(slse.reshape(B * HEADS, T), undo, axis=1)
        o = o.reshape(B * HEADS, NHASH, S, DH)
        logits = logits.reshape(B * HEADS, NHASH, S, 1)
        lmax = jnp.max(logits, axis=1, keepdims=True)
        w = jnp.exp(logits - lmax)
        w = w / jnp.sum(w, axis=1, keepdims=True)
        ao = jnp.sum(o * w, axis=1)
        ao = (ao.reshape(B, HEADS, S, DH).transpose(0, 2, 1, 3)
              .reshape(B * S, D))

        x1, x2 = _back_call(ao, x1, x2, p)

    xf = (x1 + x2) * 0.5
    xf = xf.reshape(B, S, D)
    return xf[:, 0] @ params['Wc'] + params['bc']


# speedup vs baseline: 3.3760x; 3.3760x over previous
import functools

import jax
import jax.numpy as jnp
from jax import lax
from jax.experimental import pallas as pl
from jax.experimental.pallas import tpu as pltpu
from jax.experimental.pallas import tpu_sc as plsc

HEADS = 8
BUCKET = 64
NHASH = 4
ROWBLK = 512
NW = 32


def _ln(x, g, b):
    m = jnp.mean(x, axis=-1, keepdims=True)
    v = jnp.mean(jnp.square(x - m), axis=-1, keepdims=True)
    return (x - m) * jax.lax.rsqrt(v + 1e-5) * g + b


def _qkv_body(x_ref, g_ref, b_ref, wqk_ref, wv_ref, out_ref):
    h = _ln(x_ref[...], g_ref[...], b_ref[...])
    qk = jnp.dot(h, wqk_ref[...], preferred_element_type=jnp.float32)
    v = jnp.dot(h, wv_ref[...], preferred_element_type=jnp.float32)
    out_ref[0] = jnp.concatenate([qk, v], axis=-1)


def _qkv_call(x, g, b, wqk, wv, B, S):
    R, D = x.shape
    DH = D // HEADS
    wqk_r = wqk.reshape(D, HEADS, DH).transpose(1, 0, 2).reshape(HEADS * D, DH)
    wv_r = wv.reshape(D, HEADS, DH).transpose(1, 0, 2).reshape(HEADS * D, DH)
    return pl.pallas_call(
        _qkv_body,
        grid=(B, HEADS),
        in_specs=[pl.BlockSpec((S, D), lambda i, j: (i, 0)),
                  pl.BlockSpec((1, D), lambda i, j: (0, 0)),
                  pl.BlockSpec((1, D), lambda i, j: (0, 0)),
                  pl.BlockSpec((D, DH), lambda i, j: (j, 0)),
                  pl.BlockSpec((D, DH), lambda i, j: (j, 0))],
        out_specs=pl.BlockSpec((1, S, 2 * DH), lambda i, j: (j * B + i, 0, 0)),
        out_shape=jax.ShapeDtypeStruct((HEADS * B, S, 2 * DH), jnp.float32),
    )(x, g.reshape(1, D), b.reshape(1, D), wqk_r, wv_r)


def _keys_body(qkv_ref, rot_ref, keys_ref, *, S, n_buckets, d):
    q = qkv_ref[0][:, :d]
    r = jnp.dot(q, rot_ref[...], preferred_element_type=jnp.float32)
    r = r.reshape(S, NHASH, n_buckets // 2)
    full = jnp.concatenate([r, -r], axis=-1)
    am = jnp.argmax(full, axis=-1).astype(jnp.int32)
    bucket = am + (jnp.arange(NHASH, dtype=jnp.int32) * n_buckets)[None, :]
    keys = (S * bucket).T + jnp.arange(S, dtype=jnp.int32)[None, :]
    keys_ref[...] = keys.reshape(1, 1, NHASH * S)


def _keys_call(qkvh, rot, S, n_buckets):
    bh, _, d2 = qkvh.shape
    d = d2 // 2
    return pl.pallas_call(
        functools.partial(_keys_body, S=S, n_buckets=n_buckets, d=d),
        grid=(bh,),
        in_specs=[pl.BlockSpec((1, S, d2), lambda i: (i, 0, 0)),
                  pl.BlockSpec((d, NHASH * (n_buckets // 2)), lambda i: (0, 0))],
        out_specs=pl.BlockSpec((1, 1, NHASH * S), lambda i: (i, 0, 0)),
        out_shape=jax.ShapeDtypeStruct((bh, 1, NHASH * S), jnp.int32),
    )(qkvh, rot)


def _sort_body(keys_ref, undo_ref, *, T, NB):
    SUB = 128
    NBLK = T // SUB
    k = keys_ref[0, 0]
    bucket = jax.lax.shift_right_logical(k, 9)
    oh = (bucket[:, None] == jax.lax.broadcasted_iota(jnp.int32, (1, NB), 1)
          ).astype(jnp.float32)
    tri = (jax.lax.broadcasted_iota(jnp.int32, (SUB, SUB), 0)
           >= jax.lax.broadcasted_iota(jnp.int32, (SUB, SUB), 1)
           ).astype(jnp.float32)
    incl = []
    tot = []
    for b in range(NBLK):
        blk = oh[b * SUB:(b + 1) * SUB]
        ic = jnp.dot(tri, blk, preferred_element_type=jnp.float32)
        incl.append(ic)
        tot.append(ic[SUB - 1:SUB])
    tot = jnp.concatenate(tot, axis=0)
    tri_ex = (jax.lax.broadcasted_iota(jnp.int32, (NBLK, NBLK), 0)
              > jax.lax.broadcasted_iota(jnp.int32, (NBLK, NBLK), 1)
              ).astype(jnp.float32)
    off = jnp.dot(tri_ex, tot, preferred_element_type=jnp.float32)
    cnt = off[NBLK - 1:NBLK] + tot[NBLK - 1:NBLK]
    triu = (jax.lax.broadcasted_iota(jnp.int32, (NB, NB), 0)
            < jax.lax.broadcasted_iota(jnp.int32, (NB, NB), 1)
            ).astype(jnp.float32)
    start = jnp.dot(cnt, triu, preferred_element_type=jnp.float32)
    pieces = []
    for b in range(NBLK):
        pref = incl[b] + off[b:b + 1]
        ohb = oh[b * SUB:(b + 1) * SUB]
        pieces.append(jnp.sum(ohb * (pref + start), axis=-1) - 1.0)
    undo_ref[0, 0] = jnp.concatenate(pieces, axis=0).astype(jnp.int32)


def _sort_call(keys, NB):
    bh, T = keys.shape
    iblk = pl.BlockSpec((1, 1, T), lambda i: (i, 0, 0))
    return pl.pallas_call(
        functools.partial(_sort_body, T=T, NB=NB),
        grid=(bh,),
        in_specs=[iblk],
        out_specs=iblk,
        out_shape=jax.ShapeDtypeStruct((bh, 1, T), jnp.int32),
    )(keys.reshape(bh, 1, T)).reshape(bh, T)


def _sc_mesh():
    return plsc.VectorSubcoreMesh(core_axis_name="c", subcore_axis_name="s")


def _route_in_call(qkvh, undo, S):
    bh, _, d2 = qkvh.shape
    T = undo.shape[1]
    NCH = T // 128
    CPS = S // 128
    rows_per_w = bh // NW

    @functools.partial(
        pl.kernel,
        out_type=jax.ShapeDtypeStruct((bh, T, d2), jnp.float32),
        scratch_types=[pltpu.VMEM((S, d2), jnp.float32),
                       pltpu.VMEM((NCH, 128), jnp.int32)],
        mesh=_sc_mesh(),
    )
    def run(qkv_h, undo_h, sqkv_h, qkv_v, undo_v):
        wid = lax.axis_index("s") * 2 + lax.axis_index("c")
        for rr in range(rows_per_w):
            r = wid * rows_per_w + rr
            pltpu.sync_copy(qkv_h.at[r], qkv_v)
            pltpu.sync_copy(undo_h.at[r], undo_v)

            def chunk(c, _):
                s0 = lax.rem(c, CPS) * 128
                pltpu.sync_copy(qkv_v.at[pl.ds(s0, 128), :],
                                sqkv_h.at[r].at[undo_v.at[c]])
                return 0

            lax.fori_loop(0, NCH, chunk, 0)

    return run(qkvh, undo.reshape(bh, NCH, 128))


def _route_out_call(so, undo):
    bh, T, d2 = so.shape
    NCH = T // 128
    rows_per_w = bh // NW

    @functools.partial(
        pl.kernel,
        out_type=jax.ShapeDtypeStruct((bh, T, d2), jnp.float32),
        scratch_types=[pltpu.VMEM((NCH, 128), jnp.int32),
                       pltpu.VMEM((128, d2), jnp.float32)],
        mesh=_sc_mesh(),
    )
    def run(so_h, undo_h, o_h, undo_v, rows_v):
        wid = lax.axis_index("s") * 2 + lax.axis_index("c")
        for rr in range(rows_per_w):
            r = wid * rows_per_w + rr
            pltpu.sync_copy(undo_h.at[r], undo_v)

            def chunk(c, _):
                pltpu.sync_copy(so_h.at[r].at[undo_v.at[c]], rows_v)
                pltpu.sync_copy(rows_v, o_h.at[r].at[pl.ds(c * 128, 128), :])
                return 0

            lax.fori_loop(0, NCH, chunk, 0)

    return run(so, undo.reshape(bh, NCH, 128))


def _attn_body(sqkv_ref, st_ref, so_ref, *, n_chunks, d, S):
    scale = d ** -0.5
    for c in range(n_chunks):
        p = (c - 1) % n_chunks
        qc = sqkv_ref[0, c * BUCKET:(c + 1) * BUCKET, :d]
        kp = sqkv_ref[0, p * BUCKET:(p + 1) * BUCKET, :d]
        vc = sqkv_ref[0, c * BUCKET:(c + 1) * BUCKET, d:]
        vp = sqkv_ref[0, p * BUCKET:(p + 1) * BUCKET, d:]
        tq = st_ref[0, 0, c * BUCKET:(c + 1) * BUCKET]
        tp = st_ref[0, 0, p * BUCKET:(p + 1) * BUCKET]
        kv = jnp.concatenate([qc, kp], axis=0)
        nrm = jnp.maximum(
            jnp.sqrt(jnp.sum(kv * kv, axis=-1, keepdims=True)), 1e-12)
        k = kv / nrm
        v = jnp.concatenate([vc, vp], axis=0)
        tkv = jnp.concatenate([tq, tp], axis=0)
        dots = jnp.dot(qc, k.T, preferred_element_type=jnp.float32) * scale
        dots = jnp.where(tq[:, None] == tkv[None, :], -5e4, dots)
        m = jnp.max(dots, axis=-1, keepdims=True)
        e = jnp.exp(dots - m)
        s = jnp.sum(e, axis=-1, keepdims=True)
        o = jnp.dot(e / s, v, preferred_element_type=jnp.float32)
        lse = jnp.broadcast_to(m + jnp.log(s), (BUCKET, d))
        so_ref[0, c * BUCKET:(c + 1) * BUCKET, :] = (
            jnp.concatenate([o, lse], axis=-1))


def _attn_call(sqkv, st, S):
    bh, T, d2 = sqkv.shape
    d = d2 // 2
    n_chunks = T // BUCKET
    rblk = pl.BlockSpec((1, T, d2), lambda i: (i, 0, 0))
    iblk = pl.BlockSpec((1, 1, T), lambda i: (i, 0, 0))
    return pl.pallas_call(
        functools.partial(_attn_body, n_chunks=n_chunks, d=d, S=S),
        grid=(bh,),
        in_specs=[rblk, iblk],
        out_specs=rblk,
        out_shape=jax.ShapeDtypeStruct((bh, T, d2), jnp.float32),
    )(sqkv, st.reshape(bh, 1, T))


def _comb_body(o_ref, ao_ref, *, S, d):
    ov = o_ref[0]
    o4 = ov[:, :d].reshape(NHASH, S, d)
    lse = ov[:, d].reshape(NHASH, S)
    m = jnp.max(lse, axis=0, keepdims=True)
    w = jnp.exp(lse - m)
    w = w / jnp.sum(w, axis=0, keepdims=True)
    ao_ref[0, 0] = jnp.sum(o4 * w[:, :, None], axis=0)


def _comb_call(o, B, S, D):
    bh, T, d2 = o.shape
    d = d2 // 2
    return pl.pallas_call(
        functools.partial(_comb_body, S=S, d=d),
        grid=(bh,),
        in_specs=[pl.BlockSpec((1, T, d2), lambda i: (i, 0, 0))],
        out_specs=pl.BlockSpec((1, 1, S, d), lambda i: (i // B, i % B, 0, 0)),
        out_shape=jax.ShapeDtypeStruct((HEADS, B, S, d), jnp.float32),
    )(o)


def _back_body(ao_ref, x1_ref, x2_ref, wo_ref, bo_ref, g_ref, b_ref,
               w1_ref, b1_ref, w2_ref, b2_ref, x1o_ref, x2o_ref):
    x1n = x1_ref[...] + bo_ref[...]
    for hh in range(HEADS):
        x1n = x1n + jnp.dot(ao_ref[hh, 0], wo_ref[hh],
                            preferred_element_type=jnp.float32)
    h = _ln(x1n, g_ref[...], b_ref[...])
    t = jnp.dot(h, w1_ref[...], preferred_element_type=jnp.float32) + b1_ref[...]
    t = 0.5 * t * (1.0 + jax.lax.erf(t * (2.0 ** -0.5)))
    x2o_ref[...] = x2_ref[...] + jnp.dot(
        t, w2_ref[...], preferred_element_type=jnp.float32) + b2_ref[...]
    x1o_ref[...] = x1n


def _back_call(ao, x1, x2, p):
    R, D = x1.shape
    H, B, S, DH = ao.shape
    FFD = p['W1'].shape[1]
    wo_r = p['Wo'].reshape(H, DH, D)
    grid = (R // ROWBLK,)
    blk = pl.BlockSpec((ROWBLK, D), lambda i: (i, 0))
    return pl.pallas_call(
        _back_body,
        grid=grid,
        in_specs=[pl.BlockSpec((H, 1, S, DH), lambda i: (0, i, 0, 0)),
                  blk, blk,
                  pl.BlockSpec((H, DH, D), lambda i: (0, 0, 0)),
                  pl.BlockSpec((1, D), lambda i: (0, 0)),
                  pl.BlockSpec((1, D), lambda i: (0, 0)),
                  pl.BlockSpec((1, D), lambda i: (0, 0)),
                  pl.BlockSpec((D, FFD), lambda i: (0, 0)),
                  pl.BlockSpec((1, FFD), lambda i: (0, 0)),
                  pl.BlockSpec((FFD, D), lambda i: (0, 0)),
                  pl.BlockSpec((1, D), lambda i: (0, 0))],
        out_specs=[blk, blk],
        out_shape=[jax.ShapeDtypeStruct((R, D), jnp.float32)] * 2,
    )(ao, x1, x2, wo_r, p['bo'].reshape(1, D), p['ln2_g'].reshape(1, D),
      p['ln2_b'].reshape(1, D), p['W1'], p['b1'].reshape(1, FFD),
      p['W2'], p['b2'].reshape(1, D))


def kernel(input_ids, params):
    B, S = input_ids.shape
    D = params['token_emb'].shape[1]
    DH = D // HEADS
    n_buckets = S // BUCKET
    T = NHASH * S

    x = params['token_emb'][input_ids] + params['pos_emb'][:S][None, :, :]
    x1 = x.reshape(B * S, D)
    x2 = x1

    for i, p in enumerate(params['layers']):
        rot_key = jax.random.fold_in(jax.random.key(42), i)
        rot = jax.random.normal(rot_key, (1, DH, NHASH, n_buckets // 2),
                                dtype=jnp.float32)
        rot = rot.reshape(DH, NHASH * (n_buckets // 2))

        qkvh = _qkv_call(x2, p['ln1_g'], p['ln1_b'], p['Wqk'], p['Wv'], B, S)
        keys = _keys_call(qkvh, rot, S, n_buckets).reshape(HEADS * B, T)
        undo = _sort_call(keys, NHASH * n_buckets)
        sqkv = _route_in_call(qkvh, undo, S)
        bidx = jnp.arange(HEADS * B, dtype=jnp.int32)[:, None]
        t_row = jnp.broadcast_to(jnp.arange(T, dtype=jnp.int32) & (S - 1),
                                 (HEADS * B, T))
        st = jnp.zeros((HEADS * B, T), jnp.int32).at[bidx, undo].set(t_row)
        so = _attn_call(sqkv, st, S)
        o = _route_out_call(so, undo)
        ao = _comb_call(o, B, S, D)
        x1, x2 = _back_call(ao, x1, x2, p)

    xf = (x1 + x2) * 0.5
    xf = xf.reshape(B, S, D)
    return xf[:, 0] @ params['Wc'] + params['bc']
